# Initial kernel scaffold; baseline (speedup 1.0000x reference)
import jax
import jax.numpy as jnp
from jax.experimental import pallas as pl


def kernel(x, pos, batch, sa1_params, sa2_params, sa3_params, head_params):
    raise NotImplementedError("write your pallas kernel here")


# speedup vs baseline: 4.7645x; 4.7645x over previous
import functools

import jax
import jax.numpy as jnp
from jax import lax
from jax.experimental import pallas as pl
from jax.experimental.pallas import tpu as pltpu
from jax.experimental.pallas import tpu_sc as plsc

_N0 = 1024
_K = 64
_EPS = 1e-5


def _fps_body(px_ref, py_ref, pz_ref, cx_ref, cy_ref, cz_ref, *, M):
    px = px_ref[...]
    py = py_ref[...]
    pz = pz_ref[...]
    Bb, Nn = px.shape
    iota_n = lax.broadcasted_iota(jnp.int32, (Bb, Nn), 1)
    iota_m = lax.broadcasted_iota(jnp.int32, (Bb, M), 1)

    def gather_last(last):
        oh = (iota_n == last).astype(jnp.float32)
        lx = jnp.sum(px * oh, axis=1, keepdims=True)
        ly = jnp.sum(py * oh, axis=1, keepdims=True)
        lz = jnp.sum(pz * oh, axis=1, keepdims=True)
        return lx, ly, lz

    def step(i, carry):
        dist, last, cx, cy, cz = carry
        lx, ly, lz = gather_last(last)
        cx = jnp.where(iota_m == i - 1, lx, cx)
        cy = jnp.where(iota_m == i - 1, ly, cy)
        cz = jnp.where(iota_m == i - 1, lz, cz)
        d = (px - lx) ** 2 + (py - ly) ** 2 + (pz - lz) ** 2
        dist = jnp.minimum(dist, d)
        mx = jnp.max(dist, axis=1, keepdims=True)
        nxt = jnp.min(jnp.where(dist == mx, iota_n, Nn), axis=1,
                      keepdims=True).astype(jnp.int32)
        return dist, nxt, cx, cy, cz

    dist0 = jnp.full((Bb, Nn), jnp.inf, jnp.float32)
    last0 = jnp.zeros((Bb, 1), jnp.int32)
    z = jnp.zeros((Bb, M), jnp.float32)
    dist, last, cx, cy, cz = lax.fori_loop(
        1, M, step, (dist0, last0, z, z, z))
    lx, ly, lz = gather_last(last)
    cx_ref[...] = jnp.where(iota_m == M - 1, lx, cx)
    cy_ref[...] = jnp.where(iota_m == M - 1, ly, cy)
    cz_ref[...] = jnp.where(iota_m == M - 1, lz, cz)


def _fps(px, py, pz, M):
    Bb, Nn = px.shape
    out = jax.ShapeDtypeStruct((Bb, M), jnp.float32)
    return pl.pallas_call(
        functools.partial(_fps_body, M=M),
        out_shape=(out, out, out),
    )(px, py, pz)


def _nbr_body(px_ref, py_ref, pz_ref, cx_ref, cy_ref, cz_ref,
              idx_ref, msk_ref, cnt_ref, *, r2, Nn, M):
    b = pl.program_id(0)
    px = px_ref[0]
    py = py_ref[0]
    pz = pz_ref[0]
    cx = cx_ref[0]
    cy = cy_ref[0]
    cz = cz_ref[0]
    d2 = (cx - px) ** 2 + (cy - py) ** 2 + (cz - pz) ** 2
    d2 = jnp.where(d2 <= r2, d2, jnp.inf)
    iota_n = lax.broadcasted_iota(jnp.int32, (M, Nn), 1)
    iota_k = lax.broadcasted_iota(jnp.int32, (M, _K), 1)

    def step(k, carry):
        d2c, idxs, msk = carry
        mn = jnp.min(d2c, axis=1, keepdims=True)
        j = jnp.min(jnp.where(d2c == mn, iota_n, Nn), axis=1,
                    keepdims=True).astype(jnp.int32)
        ok = (mn <= r2).astype(jnp.float32)
        idxs = jnp.where(iota_k == k, j, idxs)
        msk = jnp.where(iota_k == k, ok, msk)
        d2c = jnp.where(iota_n == j, jnp.inf, d2c)
        return d2c, idxs, msk

    idx0 = jnp.zeros((M, _K), jnp.int32)
    msk0 = jnp.zeros((M, _K), jnp.float32)
    _, idxs, msk = lax.fori_loop(0, _K, step, (d2, idx0, msk0))
    idx_ref[0] = idxs + b * Nn
    msk_ref[0] = msk

    @pl.when(b == 0)
    def _():
        cnt_ref[...] = jnp.zeros_like(cnt_ref)

    cnt_ref[...] += jnp.sum(msk).reshape(1, 1)


def _nbr(px, py, pz, cx, cy, cz, r):
    Bb, Nn = px.shape
    M = cx.shape[1]
    px3 = px.reshape(Bb, 1, Nn)
    py3 = py.reshape(Bb, 1, Nn)
    pz3 = pz.reshape(Bb, 1, Nn)
    cx3 = cx.reshape(Bb, M, 1)
    cy3 = cy.reshape(Bb, M, 1)
    cz3 = cz.reshape(Bb, M, 1)
    row = pl.BlockSpec((1, 1, Nn), lambda b: (b, 0, 0))
    col = pl.BlockSpec((1, M, 1), lambda b: (b, 0, 0))
    return pl.pallas_call(
        functools.partial(_nbr_body, r2=r * r, Nn=Nn, M=M),
        grid=(Bb,),
        in_specs=[row, row, row, col, col, col],
        out_specs=(
            pl.BlockSpec((1, M, _K), lambda b: (b, 0, 0)),
            pl.BlockSpec((1, M, _K), lambda b: (b, 0, 0)),
            pl.BlockSpec((1, 1), lambda b: (0, 0)),
        ),
        out_shape=(
            jax.ShapeDtypeStruct((Bb, M, _K), jnp.int32),
            jax.ShapeDtypeStruct((Bb, M, _K), jnp.float32),
            jax.ShapeDtypeStruct((1, 1), jnp.float32),
        ),
    )(px3, py3, pz3, cx3, cy3, cz3)


def _sc_gather(table, idx, chunk):
    E = idx.shape[0]
    D = table.shape[1]
    info = plsc.get_sparse_core_info()
    nw = info.num_cores * info.num_subcores
    b_per_w = E // nw
    n_chunks = b_per_w // chunk
    mesh = plsc.VectorSubcoreMesh(core_axis_name="c", subcore_axis_name="s")

    @functools.partial(
        pl.kernel,
        mesh=mesh,
        compiler_params=pltpu.CompilerParams(use_tc_tiling_on_sc=False),
        out_type=jax.ShapeDtypeStruct((E, D), jnp.float32),
        scratch_types=[
            pltpu.VMEM((chunk,), jnp.int32),
            pltpu.VMEM((chunk, D), jnp.float32),
            pltpu.SemaphoreType.DMA,
        ],
    )
    def gather_kernel(table_hbm, idx_hbm, out_hbm, idx_v, rows_v, sem):
        wid = lax.axis_index("s") * info.num_cores + lax.axis_index("c")
        base = wid * b_per_w
        n_sub = chunk // 128

        def body(i, carry):
            off = base + i * chunk
            pltpu.sync_copy(idx_hbm.at[pl.ds(off, chunk)], idx_v)
            copies = []
            for j in range(n_sub):
                sl = pl.ds(j * 128, 128)
                copies.append(pltpu.async_copy(
                    table_hbm.at[idx_v.at[sl]], rows_v.at[sl], sem))
            for c in copies:
                c.wait()
            pltpu.sync_copy(rows_v, out_hbm.at[pl.ds(off, chunk)])
            return carry

        lax.fori_loop(0, n_chunks, body, 0)

    return gather_kernel(table, idx)


def _bf(v):
    return v.astype(jnp.bfloat16)


def _l1_body(g_ref, cen_ref, w_ref, b_ref, m_ref,
             h_ref, s_ref, *, Rc):
    i = pl.program_id(0)
    g = g_ref[...]
    cen = cen_ref[...]
    Dp = g.shape[1]
    msg = g - jnp.broadcast_to(cen[:, None, :], (Rc, _K, Dp)).reshape(
        Rc * _K, Dp)
    h = jnp.dot(_bf(msg), _bf(w_ref[...]),
                preferred_element_type=jnp.float32)
    h = jnp.maximum(h + b_ref[...], 0.0)
    h_ref[...] = h
    hm = h * m_ref[...]

    @pl.when(i == 0)
    def _():
        s_ref[...] = jnp.zeros_like(s_ref)

    s_ref[...] += jnp.sum(hm, axis=0, keepdims=True)


def _layer1(g, cen_tab, w, b, msk, chunk):
    E, Dp = g.shape
    C = w.shape[1]
    Rc = chunk // _K
    grid = (E // chunk,)
    return pl.pallas_call(
        functools.partial(_l1_body, Rc=Rc),
        grid=grid,
        in_specs=[
            pl.BlockSpec((chunk, Dp), lambda i: (i, 0)),
            pl.BlockSpec((Rc, Dp), lambda i: (i, 0)),
            pl.BlockSpec((Dp, C), lambda i: (0, 0)),
            pl.BlockSpec((1, C), lambda i: (0, 0)),
            pl.BlockSpec((chunk, 1), lambda i: (i, 0)),
        ],
        out_specs=(
            pl.BlockSpec((chunk, C), lambda i: (i, 0)),
            pl.BlockSpec((1, C), lambda i: (0, 0)),
        ),
        out_shape=(
            jax.ShapeDtypeStruct((E, C), jnp.float32),
            jax.ShapeDtypeStruct((1, C), jnp.float32),
        ),
    )(g, cen_tab, w, b, msk)


def _varpass_body(h_ref, mu_ref, m_ref, v_ref):
    i = pl.program_id(0)
    d = h_ref[...] - mu_ref[...]

    @pl.when(i == 0)
    def _():
        v_ref[...] = jnp.zeros_like(v_ref)

    v_ref[...] += jnp.sum(d * d * m_ref[...], axis=0, keepdims=True)


def _varpass(h, mu, msk, chunk):
    E, C = h.shape
    grid = (E // chunk,)
    return pl.pallas_call(
        _varpass_body,
        grid=grid,
        in_specs=[
            pl.BlockSpec((chunk, C), lambda i: (i, 0)),
            pl.BlockSpec((1, C), lambda i: (0, 0)),
            pl.BlockSpec((chunk, 1), lambda i: (i, 0)),
        ],
        out_specs=pl.BlockSpec((1, C), lambda i: (0, 0)),
        out_shape=jax.ShapeDtypeStruct((1, C), jnp.float32),
    )(h, mu, msk)


def _lmid_body(x_ref, mu_ref, sd_ref, ga_ref, be_ref, w_ref, b_ref, m_ref,
               h_ref, s_ref):
    i = pl.program_id(0)
    hn = (x_ref[...] - mu_ref[...]) / sd_ref[...] * ga_ref[...] + be_ref[...]
    h = jnp.dot(_bf(hn), _bf(w_ref[...]),
                preferred_element_type=jnp.float32)
    h = jnp.maximum(h + b_ref[...], 0.0)
    h_ref[...] = h
    hm = h * m_ref[...]

    @pl.when(i == 0)
    def _():
        s_ref[...] = jnp.zeros_like(s_ref)

    s_ref[...] += jnp.sum(hm, axis=0, keepdims=True)


def _layer(x, mu, sd, ga, be, w, b, msk, chunk):
    E, D = x.shape
    C = w.shape[1]
    grid = (E // chunk,)
    vec = pl.BlockSpec((1, D), lambda i: (0, 0))
    return pl.pallas_call(
        _lmid_body,
        grid=grid,
        in_specs=[
            pl.BlockSpec((chunk, D), lambda i: (i, 0)),
            vec, vec, vec, vec,
            pl.BlockSpec((D, C), lambda i: (0, 0)),
            pl.BlockSpec((1, C), lambda i: (0, 0)),
            pl.BlockSpec((chunk, 1), lambda i: (i, 0)),
        ],
        out_specs=(
            pl.BlockSpec((chunk, C), lambda i: (i, 0)),
            pl.BlockSpec((1, C), lambda i: (0, 0)),
        ),
        out_shape=(
            jax.ShapeDtypeStruct((E, C), jnp.float32),
            jax.ShapeDtypeStruct((1, C), jnp.float32),
        ),
    )(x, mu, sd, ga, be, w, b, msk)


def _pool_body(h_ref, m_ref, mu_ref, sd_ref, ga_ref, be_ref, o_ref, *, Rc):
    h = h_ref[...]
    C = h.shape[1]
    hn = (h - mu_ref[...]) / sd_ref[...] * ga_ref[...] + be_ref[...]
    hn = jnp.where(m_ref[...] > 0.0, hn, -jnp.inf)
    o_ref[...] = jnp.max(hn.reshape(Rc, _K, C), axis=1)


def _pool(h, msk, mu, sd, ga, be, chunk):
    E, C = h.shape
    Rc = chunk // _K
    grid = (E // chunk,)
    vec = pl.BlockSpec((1, C), lambda i: (0, 0))
    return pl.pallas_call(
        functools.partial(_pool_body, Rc=Rc),
        grid=grid,
        in_specs=[
            pl.BlockSpec((chunk, C), lambda i: (i, 0)),
            pl.BlockSpec((chunk, 1), lambda i: (i, 0)),
            vec, vec, vec, vec,
        ],
        out_specs=pl.BlockSpec((Rc, C), lambda i: (i, 0)),
        out_shape=jax.ShapeDtypeStruct((E // _K, C), jnp.float32),
    )(h, msk, mu, sd, ga, be)


def _gpool_body(h_ref, mu_ref, sd_ref, ga_ref, be_ref, o_ref):
    hn = (h_ref[...] - mu_ref[...]) / sd_ref[...] * ga_ref[...] + be_ref[...]
    o_ref[0] = jnp.max(hn, axis=0, keepdims=True)


def _gpool(h, mu, sd, ga, be, rows):
    E, C = h.shape
    grid = (E // rows,)
    vec = pl.BlockSpec((1, C), lambda i: (0, 0))
    out = pl.pallas_call(
        _gpool_body,
        grid=grid,
        in_specs=[
            pl.BlockSpec((rows, C), lambda i: (i, 0)),
            vec, vec, vec, vec,
        ],
        out_specs=pl.BlockSpec((1, 1, C), lambda i: (i, 0, 0)),
        out_shape=jax.ShapeDtypeStruct((E // rows, 1, C), jnp.float32),
    )(h, mu, sd, ga, be)
    return out.reshape(E // rows, C)


def _stats(h, s, cnt, msk, chunk):
    mu = s / cnt
    v = _varpass(h, mu, msk, chunk)
    return mu, jnp.sqrt(v / cnt + _EPS)


def _sa_stage(table, cen_tab, idx_flat, msk_flat, cnt, params,
              chunk, gchunk):
    gt = _sc_gather(table, idx_flat, gchunk)
    (w1, b1, g1, be1), (w2, b2, g2, be2), (w3, b3, g3, be3) = params
    Dp = table.shape[1]
    w1p = jnp.zeros((Dp, w1.shape[1]), jnp.float32).at[:w1.shape[0]].set(w1)
    h1, s1 = _layer1(gt, cen_tab, w1p, b1.reshape(1, -1), msk_flat, chunk)
    mu1, sd1 = _stats(h1, s1, cnt, msk_flat, chunk)
    h2, s2 = _layer(h1, mu1, sd1, g1.reshape(1, -1), be1.reshape(1, -1),
                    w2, b2.reshape(1, -1), msk_flat, chunk)
    mu2, sd2 = _stats(h2, s2, cnt, msk_flat, chunk)
    h3, s3 = _layer(h2, mu2, sd2, g2.reshape(1, -1), be2.reshape(1, -1),
                    w3, b3.reshape(1, -1), msk_flat, chunk)
    mu3, sd3 = _stats(h3, s3, cnt, msk_flat, chunk)
    return _pool(h3, msk_flat, mu3, sd3, g3.reshape(1, -1),
                 be3.reshape(1, -1), chunk)


def kernel(x, pos, batch, sa1_params, sa2_params, sa3_params, head_params):
    Bb = batch.shape[0] // _N0
    Nn = x.shape[0] // Bb
    M1, M2 = Nn // 2, Nn // 8

    pos3 = pos.reshape(Bb, Nn, 3)
    px, py, pz = pos3[..., 0], pos3[..., 1], pos3[..., 2]

    c1x, c1y, c1z = _fps(px, py, pz, M1)
    idx1, msk1, cnt1 = _nbr(px, py, pz, c1x, c1y, c1z, 0.1)
    cnt1 = jnp.maximum(cnt1, 1.0)
    table1 = jnp.concatenate(
        [x.reshape(Bb * Nn, 1), pos,
         jnp.zeros((Bb * Nn, 12), jnp.float32)], axis=1)
    cen1 = jnp.stack([c1x, c1y, c1z], axis=-1).reshape(Bb * M1, 3)
    cen_tab1 = jnp.zeros((Bb * M1, 16), jnp.float32).at[:, 1:4].set(cen1)
    x1 = _sa_stage(table1, cen_tab1, idx1.reshape(-1),
                   msk1.reshape(-1, 1), cnt1, sa1_params,
                   chunk=4096, gchunk=2048)

    c2x, c2y, c2z = _fps(c1x, c1y, c1z, M2)
    idx2, msk2, cnt2 = _nbr(c1x, c1y, c1z, c2x, c2y, c2z, 0.2)
    cnt2 = jnp.maximum(cnt2, 1.0)
    table2 = jnp.concatenate(
        [x1, cen1, jnp.zeros((Bb * M1, 13), jnp.float32)], axis=1)
    cen2 = jnp.stack([c2x, c2y, c2z], axis=-1).reshape(Bb * M2, 3)
    cen_tab2 = jnp.zeros((Bb * M2, 144), jnp.float32).at[:, 128:131].set(
        cen2)
    x2 = _sa_stage(table2, cen_tab2, idx2.reshape(-1),
                   msk2.reshape(-1, 1), cnt2, sa2_params,
                   chunk=4096, gchunk=512)

    (w1, b1, g1, be1), (w2, b2, g2, be2), (w3, b3, g3, be3) = sa3_params
    E3 = Bb * M2
    table3 = jnp.concatenate(
        [x2, cen2, jnp.zeros((E3, 384 - 259), jnp.float32)], axis=1)
    ones = jnp.ones((E3, 1), jnp.float32)
    w1p = jnp.zeros((384, w1.shape[1]), jnp.float32).at[:259].set(w1)
    zid = jnp.zeros((1, 384), jnp.float32)
    oid = jnp.ones((1, 384), jnp.float32)
    cnt3 = jnp.float32(E3)
    h1, s1 = _layer(table3, zid, oid, oid, zid,
                    w1p, b1.reshape(1, -1), ones, E3 // 2)
    mu1, sd1 = _stats(h1, s1, cnt3, ones, E3 // 2)
    h2, s2 = _layer(h1, mu1, sd1, g1.reshape(1, -1), be1.reshape(1, -1),
                    w2, b2.reshape(1, -1), ones, E3 // 2)
    mu2, sd2 = _stats(h2, s2, cnt3, ones, E3 // 2)
    h3, s3 = _layer(h2, mu2, sd2, g2.reshape(1, -1), be2.reshape(1, -1),
                    w3, b3.reshape(1, -1), ones, E3 // 2)
    mu3, sd3 = _stats(h3, s3, cnt3, ones, E3 // 2)
    xg = _gpool(h3, mu3, sd3, g3.reshape(1, -1), be3.reshape(1, -1), M2)

    return _head_full(xg, head_params)


def _head_full(xg, head_params):
    (w1, b1, g1, be1), (w2, b2, g2, be2), (w3, b3, g3, be3), (w4, b4) = \
        head_params
    return pl.pallas_call(
        _head_body2,
        out_shape=jax.ShapeDtypeStruct((xg.shape[0], 1), jnp.float32),
    )(xg, w1, b1.reshape(1, -1), g1.reshape(1, -1), be1.reshape(1, -1),
      w2, b2.reshape(1, -1), g2.reshape(1, -1), be2.reshape(1, -1),
      w3, b3.reshape(1, -1), g3.reshape(1, -1), be3.reshape(1, -1),
      w4, b4.reshape(1, 1))


def _head_body2(x_ref, w1_ref, b1_ref, g1_ref, be1_ref,
                w2_ref, b2_ref, g2_ref, be2_ref,
                w3_ref, b3_ref, g3_ref, be3_ref,
                w4_ref, b4_ref, o_ref):
    def bn_relu(h, g, be):
        mean = jnp.mean(h, axis=0, keepdims=True)
        var = jnp.mean((h - mean) ** 2, axis=0, keepdims=True)
        hn = (h - mean) / jnp.sqrt(var + _EPS) * g + be
        return jnp.maximum(hn, 0.0)

    h = x_ref[...]
    h = bn_relu(jnp.dot(_bf(h), _bf(w1_ref[...]),
                        preferred_element_type=jnp.float32) + b1_ref[...],
                g1_ref[...], be1_ref[...])
    h = bn_relu(jnp.dot(_bf(h), _bf(w2_ref[...]),
                        preferred_element_type=jnp.float32) + b2_ref[...],
                g2_ref[...], be2_ref[...])
    h = bn_relu(jnp.dot(_bf(h), _bf(w3_ref[...]),
                        preferred_element_type=jnp.float32) + b3_ref[...],
                g3_ref[...], be3_ref[...])
    o_ref[...] = jnp.dot(_bf(h), _bf(w4_ref[...]),
                         preferred_element_type=jnp.float32) + b4_ref[...]
